# baseline (speedup 1.0000x reference)
import jax
import jax.numpy as jnp
from jax import lax
from jax.experimental import pallas as pl
from jax.experimental.pallas import tpu as pltpu
from jax.experimental.pallas import tpu_sc as plsc

_NC = 2
_NS = 16
_L = 16
_NW = _NC * _NS

_N = 1048576
_PER_W = _N // _NW
_NCHUNK = 4
_CHUNK = _PER_W // _NCHUNK
_CSTEPS = _CHUNK // _L
_NBINS = 64


def _sc_body(cards_hbm, rank_hbm, suit_hbm, out_hbm, cards_v, hist_v, rank_v,
             suit_v, part_v, sems, tsem):
  wid = lax.axis_index("s") * _NC + lax.axis_index("c")
  base = wid * _PER_W

  copies = []
  for q in range(_NCHUNK):
    copies.append(
        pltpu.async_copy(
            cards_hbm.at[pl.ds(base + q * _CHUNK, _CHUNK)],
            cards_v.at[pl.ds(q * _CHUNK, _CHUNK)],
            sems.at[q],
        ))
  rank_copy = pltpu.async_copy(rank_hbm, rank_v, tsem)
  suit_copy = pltpu.async_copy(suit_hbm, suit_v, tsem)

  zeros16 = jnp.zeros((_L,), jnp.float32)
  for c in range(2 * _NBINS):
    hist_v[c, pl.ds(0, _L)] = zeros16

  lane = lax.iota(jnp.int32, _L)
  ones = jnp.ones((_L,), jnp.float32)
  bank1 = jnp.int32(_NBINS)

  for q in range(_NCHUNK):
    copies[q].wait()

    @plsc.parallel_loop(q * _CSTEPS, (q + 1) * _CSTEPS, step=2, unroll=8)
    def _(i):
      c0 = cards_v[pl.ds(i * _L, _L)]
      c1 = cards_v[pl.ds((i + 1) * _L, _L)]
      plsc.addupdate_scatter(hist_v, [c0, lane], ones)
      plsc.addupdate_scatter(hist_v, [c1 + bank1, lane], ones)

  rank_copy.wait()
  suit_copy.wait()

  inv_n = jnp.float32(1.0 / _N)
  rankvec = jnp.zeros((_L,), jnp.float32)
  srows = [jnp.zeros((_L,), jnp.float32) for _ in range(4)]
  for r in range(13):
    rrow = jnp.zeros((_L,), jnp.float32)
    for s in range(4):
      h = (hist_v[13 * s + r, pl.ds(0, _L)] +
           hist_v[_NBINS + 13 * s + r, pl.ds(0, _L)])
      rrow = rrow + h
      srows[s] = srows[s] + h
    cnt = jnp.sum(rrow)
    rankvec = rankvec + cnt * (lane == r).astype(jnp.float32)
  rankvec = rankvec * inv_n

  part = jnp.zeros((_L,), jnp.float32)
  lanem = jnp.minimum(lane, 12)
  m13 = (lane < 13).astype(jnp.float32)
  for j in range(8):
    col = plsc.load_gather(rank_v, [lane * 0 + j, lanem])
    dot = jnp.sum(rankvec * col * m13)
    onehot = (lane == j).astype(jnp.float32)
    part = part + dot * onehot
  m8_11 = ((lane >= 8) & (lane < 12)).astype(jnp.float32)
  lane4 = jnp.clip(lane - 8, 0, 3)
  for s in range(4):
    scnt = jnp.sum(srows[s]) * inv_n
    row = plsc.load_gather(suit_v, [lane * 0 + s, lane4])
    part = part + scnt * row * m8_11
  part_v[pl.ds(0, _L)] = part
  pltpu.sync_copy(part_v, out_hbm.at[wid])


@jax.jit
def _sc_hist(cards, rank_embed, suit_embed):
  mesh = plsc.VectorSubcoreMesh(
      core_axis_name="c", subcore_axis_name="s", num_cores=_NC,
      num_subcores=_NS)
  return pl.kernel(
      _sc_body,
      out_type=jax.ShapeDtypeStruct((_NW, _L), jnp.float32),
      mesh=mesh,
      scratch_types=[
          pltpu.VMEM((_PER_W,), jnp.int32),
          pltpu.VMEM((2 * _NBINS, _L), jnp.float32),
          pltpu.VMEM((8, 13), jnp.float32),
          pltpu.VMEM((4, 4), jnp.float32),
          pltpu.VMEM((_L,), jnp.float32),
          pltpu.SemaphoreType.DMA((_NCHUNK,)),
          pltpu.SemaphoreType.DMA,
      ],
      compiler_params=pltpu.CompilerParams(needs_layout_passes=False),
  )(cards, rank_embed, suit_embed)


def kernel(cards, rank_embed, suit_embed):
  cards = cards.astype(jnp.int32)
  partials = _sc_hist(cards, rank_embed.T, suit_embed)
  return partials[:, :12].sum(axis=0)


# speedup vs baseline: 1.0157x; 1.0157x over previous
import jax
import jax.numpy as jnp
from jax import lax
from jax.experimental import pallas as pl
from jax.experimental.pallas import tpu as pltpu
from jax.experimental.pallas import tpu_sc as plsc

_NC = 2
_NS = 16
_L = 16
_NW = _NC * _NS

_N = 1048576
_PER_W = _N // _NW
_NCHUNK = 4
_CHUNK = _PER_W // _NCHUNK
_CSTEPS = _CHUNK // _L
_NBINS = 64


def _sc_body(cards_hbm, rank_hbm, suit_hbm, out_hbm, cards_v, hist_v, rank_v,
             suit_v, part_v, sems, tsem):
  wid = lax.axis_index("s") * _NC + lax.axis_index("c")
  base = wid * _PER_W

  copies = []
  for q in range(_NCHUNK):
    copies.append(
        pltpu.async_copy(
            cards_hbm.at[pl.ds(base + q * _CHUNK, _CHUNK)],
            cards_v.at[pl.ds(q * _CHUNK, _CHUNK)],
            sems.at[q],
        ))
  rank_copy = pltpu.async_copy(rank_hbm, rank_v, tsem)
  suit_copy = pltpu.async_copy(suit_hbm, suit_v, tsem)

  zeros16 = jnp.zeros((_L,), jnp.float32)
  for c in range(_NBINS):
    hist_v[c, pl.ds(0, _L)] = zeros16

  lane = lax.iota(jnp.int32, _L)
  ones = jnp.ones((_L,), jnp.float32)

  for q in range(_NCHUNK):
    copies[q].wait()

    @plsc.parallel_loop(q * _CSTEPS, (q + 1) * _CSTEPS, step=1, unroll=16)
    def _(i):
      c = cards_v[pl.ds(i * _L, _L)]
      plsc.addupdate_scatter(hist_v, [c, lane], ones)

  rank_copy.wait()
  suit_copy.wait()

  inv_n = jnp.float32(1.0 / _N)
  rankvec = jnp.zeros((_L,), jnp.float32)
  srows = [jnp.zeros((_L,), jnp.float32) for _ in range(4)]
  for r in range(13):
    rrow = jnp.zeros((_L,), jnp.float32)
    for s in range(4):
      h = hist_v[13 * s + r, pl.ds(0, _L)]
      rrow = rrow + h
      srows[s] = srows[s] + h
    cnt = jnp.sum(rrow)
    rankvec = rankvec + cnt * (lane == r).astype(jnp.float32)
  rankvec = rankvec * inv_n

  part = jnp.zeros((_L,), jnp.float32)
  lanem = jnp.minimum(lane, 12)
  m13 = (lane < 13).astype(jnp.float32)
  for j in range(8):
    col = plsc.load_gather(rank_v, [lane * 0 + j, lanem])
    dot = jnp.sum(rankvec * col * m13)
    onehot = (lane == j).astype(jnp.float32)
    part = part + dot * onehot
  m8_11 = ((lane >= 8) & (lane < 12)).astype(jnp.float32)
  lane4 = jnp.clip(lane - 8, 0, 3)
  for s in range(4):
    scnt = jnp.sum(srows[s]) * inv_n
    row = plsc.load_gather(suit_v, [lane * 0 + s, lane4])
    part = part + scnt * row * m8_11
  part_v[pl.ds(0, _L)] = part
  pltpu.sync_copy(part_v, out_hbm.at[wid])


@jax.jit
def _sc_hist(cards, rank_embed, suit_embed):
  mesh = plsc.VectorSubcoreMesh(
      core_axis_name="c", subcore_axis_name="s", num_cores=_NC,
      num_subcores=_NS)
  return pl.kernel(
      _sc_body,
      out_type=jax.ShapeDtypeStruct((_NW, _L), jnp.float32),
      mesh=mesh,
      scratch_types=[
          pltpu.VMEM((_PER_W,), jnp.int32),
          pltpu.VMEM((_NBINS, _L), jnp.float32),
          pltpu.VMEM((8, 13), jnp.float32),
          pltpu.VMEM((4, 4), jnp.float32),
          pltpu.VMEM((_L,), jnp.float32),
          pltpu.SemaphoreType.DMA((_NCHUNK,)),
          pltpu.SemaphoreType.DMA,
      ],
      compiler_params=pltpu.CompilerParams(needs_layout_passes=False),
  )(cards, rank_embed, suit_embed)


def kernel(cards, rank_embed, suit_embed):
  cards = cards.astype(jnp.int32)
  partials = _sc_hist(cards, rank_embed.T, suit_embed)
  return partials[:, :12].sum(axis=0)
